# Initial kernel scaffold; baseline (speedup 1.0000x reference)
import jax
import jax.numpy as jnp
from jax.experimental import pallas as pl


def kernel(indices, indptr, data):
    raise NotImplementedError("write your pallas kernel here")


# speedup vs baseline: 26.8736x; 26.8736x over previous
import dataclasses
import functools

import jax
import jax.numpy as jnp
from jax import lax
from jax.experimental import pallas as pl
from jax.experimental.pallas import tpu as pltpu
from jax.experimental.pallas import tpu_sc as plsc

L = 16
NC = 2
NS = 16
NW = NC * NS


def _sc_transpose_csr(n_rows, nnz, nnz_pad, ptr_pad, keys_hbm, ptrs_hbm,
                      data_hbm):
  chunk = nnz_pad // NW
  vpc = chunk // L
  bins_per_owner = n_rows // NS
  jvecs = bins_per_owner // L
  nptr_vecs = ptr_pad // L

  mesh = plsc.VectorSubcoreMesh(core_axis_name="c", subcore_axis_name="s",
                                num_cores=NC, num_subcores=NS)
  cp = pltpu.CompilerParams()
  if "needs_layout_passes" in pltpu.CompilerParams.__dataclass_fields__:
    cp = dataclasses.replace(cp, needs_layout_passes=False)

  @functools.partial(
      pl.kernel,
      compiler_params=cp,
      out_type=(
          jax.ShapeDtypeStruct((nnz_pad,), jnp.int32),
          jax.ShapeDtypeStruct((ptr_pad,), jnp.int32),
          jax.ShapeDtypeStruct((nnz_pad,), jnp.int32),
          jax.ShapeDtypeStruct((nnz_pad,), jnp.float32),
          jax.ShapeDtypeStruct((NW, n_rows), jnp.int32),
          jax.ShapeDtypeStruct((NW, n_rows), jnp.int32),
          jax.ShapeDtypeStruct((NS, L), jnp.int32),
      ),
      mesh=mesh,
      scratch_types=[
          pltpu.VMEM((chunk,), jnp.int32),
          pltpu.VMEM((n_rows,), jnp.int32),
          pltpu.VMEM((ptr_pad,), jnp.int32),
          pltpu.VMEM((chunk,), jnp.int32),
          pltpu.VMEM((chunk,), jnp.int32),
          pltpu.VMEM((chunk,), jnp.int32),
          pltpu.VMEM((chunk,), jnp.float32),
          pltpu.VMEM((NW, bins_per_owner), jnp.int32),
          pltpu.VMEM((bins_per_owner,), jnp.int32),
          pltpu.VMEM((L,), jnp.int32),
          pltpu.VMEM((L,), jnp.int32),
          pltpu.VMEM((L,), jnp.int32),
          pltpu.VMEM((NS, L), jnp.int32),
          pltpu.SemaphoreType.DMA,
          pltpu.SemaphoreType.DMA,
      ],
  )
  def sc_kernel(keys_h, ptrs_h, data_h,
                tind_h, tptr_h, perm_h, tdata_h, h_hbm, s_hbm, rt_hbm,
                keys, cnt, ptrs, rowh, pos, permb, datab,
                own_h, titv, accv, carryv, rtv, rt_all,
                sem, dsem):
    cid = lax.axis_index("c")
    sid = lax.axis_index("s")
    g_own = sid * NC + cid
    g_other = sid * NC + (1 - cid)

    zv = jnp.zeros((L,), jnp.int32)

    cz, _ = plsc.scan_count(jnp.zeros((L,), jnp.int32))
    alpha = jnp.min(cz)

    data_cp = pltpu.async_copy(data_h.at[g_own], datab, dsem)

    def zero_cnt():
      @pl.loop(0, n_rows // L)
      def _(i):
        cnt[pl.ds(i * L, L)] = zv

    def histogram():
      @pl.loop(0, vpc)
      def _(i):
        v = keys[pl.ds(i * L, L)]
        c, last = plsc.scan_count(v)
        base = plsc.load_gather(cnt, [v])
        plsc.store_scatter(cnt, [v], base + (c - alpha + 1), mask=last)

    pltpu.sync_copy(keys_h.at[g_other], keys)
    zero_cnt()
    histogram()
    pltpu.sync_copy(cnt, h_hbm.at[g_other])

    pltpu.sync_copy(keys_h.at[g_own], keys)
    zero_cnt()
    histogram()
    pltpu.sync_copy(cnt, h_hbm.at[g_own])

    pltpu.sync_copy(ptrs_h, ptrs)
    cb = g_own * chunk

    @pl.loop(0, vpc)
    def _(i):
      rowh[pl.ds(i * L, L)] = zv

    accv[...] = zv
    @pl.loop(0, nptr_vecs)
    def _(i):
      v = ptrs[pl.ds(i * L, L)]
      in_chunk = (v >= cb) & (v < cb + chunk)
      accv[...] = accv[...] + plsc.all_reduce_population_count(v < cb)
      c2, last2 = plsc.scan_count(v, in_chunk)
      q = jnp.where(in_chunk, v - cb, 0)
      mg = in_chunk & last2
      base = plsc.load_gather(rowh, [q])
      plsc.store_scatter(rowh, [q], base + (c2 - alpha + 1), mask=mg)

    carryv[...] = accv[...]
    @pl.loop(0, vpc)
    def _(i):
      v = rowh[pl.ds(i * L, L)]
      incl = plsc.cumsum(v) + jnp.min(carryv[...])
      rid = jnp.minimum(jnp.maximum(incl - 1, 0), n_rows - 1)
      rowh[pl.ds(i * L, L)] = rid
      carryv[...] = carryv[...] + jnp.sum(v)

    plsc.subcore_barrier()
    bin0 = sid * bins_per_owner

    cps = [pltpu.async_copy(h_hbm.at[w, pl.ds(bin0, bins_per_owner)],
                            own_h.at[w], sem) for w in range(NW)]
    for c in cps:
      c.wait()

    accv[...] = zv
    @pl.loop(0, jvecs)
    def _(j):
      tv = zv
      for w in range(NW):
        tv = tv + own_h[w, pl.ds(j * L, L)]
      titv[pl.ds(j * L, L)] = tv
      accv[...] = accv[...] + jnp.sum(tv)

    rtv[...] = accv[...]
    pltpu.sync_copy(rtv, rt_hbm.at[sid])
    plsc.subcore_barrier()
    pltpu.sync_copy(rt_hbm, rt_all)

    basev = zv
    for o2 in range(NS):
      basev = basev + rt_all[o2] * (o2 < sid).astype(jnp.int32)
    carryv[...] = basev

    @pl.loop(0, jvecs)
    def _(j):
      tv = titv[pl.ds(j * L, L)]
      excl = plsc.cumsum(tv) - tv + jnp.min(carryv[...])
      titv[pl.ds(j * L, L)] = excl
      carryv[...] = carryv[...] + jnp.sum(tv)
      run = excl
      for w in range(NW):
        h = own_h[w, pl.ds(j * L, L)]
        own_h[w, pl.ds(j * L, L)] = run
        run = run + h

    cps = [pltpu.async_copy(own_h.at[w],
                            s_hbm.at[w, pl.ds(bin0, bins_per_owner)], sem)
           for w in range(NW)]
    for c in cps:
      c.wait()

    @pl.when(cid == 0)
    def _():
      pltpu.sync_copy(titv, tptr_h.at[pl.ds(bin0, bins_per_owner)])

    @pl.when((cid == 0) & (sid == 0))
    def _():
      rtv[...] = jnp.full((L,), nnz, jnp.int32)
      pltpu.sync_copy(rtv, tptr_h.at[pl.ds(n_rows, L)])

    plsc.subcore_barrier()

    pltpu.sync_copy(s_hbm.at[g_own], cnt)

    @pl.loop(0, vpc)
    def _(i):
      v = keys[pl.ds(i * L, L)]
      c3, last3 = plsc.scan_count(v)
      base = plsc.load_gather(cnt, [v])
      rank = c3 - alpha
      pos[pl.ds(i * L, L)] = base + rank
      plsc.store_scatter(cnt, [v], base + rank + 1, mask=last3)
      permb[pl.ds(i * L, L)] = cb + i * L + lax.iota(jnp.int32, L)

    data_cp.wait()
    cp1 = pltpu.async_copy(permb, perm_h.at[pos], sem)
    cp2 = pltpu.async_copy(rowh, tind_h.at[pos], sem)
    cp3 = pltpu.async_copy(datab, tdata_h.at[pos], sem)
    cp1.wait()
    cp2.wait()
    cp3.wait()

  return sc_kernel(keys_hbm, ptrs_hbm, data_hbm)


def kernel(indices, indptr, data):
  nnz = indices.shape[0]
  n_rows = indptr.shape[0] - 1

  chunk = -(-nnz // (NW * 128)) * 128
  nnz_pad = NW * chunk
  pad = nnz_pad - nnz
  ptr_pad = -(-(n_rows + 1 + L) // L) * L

  keys_pad = jnp.concatenate(
      [indices.astype(jnp.int32),
       jnp.full((pad,), n_rows - 1, jnp.int32)]).reshape(NW, chunk)
  data_pad = jnp.concatenate(
      [data, jnp.zeros((pad,), jnp.float32)]).reshape(NW, chunk)
  ptrs_pad = jnp.concatenate(
      [indptr.astype(jnp.int32),
       jnp.full((ptr_pad - (n_rows + 1),), jnp.int32(2**30), jnp.int32)])

  t_ind, t_ptr, perm, t_data, _, _, _ = _sc_transpose_csr(
      n_rows, nnz, nnz_pad, ptr_pad, keys_pad, ptrs_pad, data_pad)
  return (t_ind[:nnz], t_ptr[:n_rows + 1], perm[:nnz], t_data[:nnz])
